# baseline (speedup 1.0000x reference)
import jax
import jax.numpy as jnp
from jax.experimental import pallas as pl
from jax.experimental.pallas import tpu as pltpu

_K = 1024
_C = 256
_TB = 1024
_N_TOK = 8192


def _vq_body(z_ref, e_ref, e2_ref, zq_ref, idx_ref, dsum_ref):
    zb = z_ref[...]
    e = e_ref[...]
    s = jnp.sum(zb * zb, axis=1, keepdims=True)
    e2 = e2_ref[0, :]
    m = jax.lax.dot_general(
        zb, e, (((1,), (1,)), ((), ())),
        preferred_element_type=jnp.float32)
    d = (s + e2[None, :]) - 2.0 * m
    minv = jnp.min(d, axis=1, keepdims=True)
    iota_f = jax.lax.broadcasted_iota(
        jnp.int32, d.shape, 1).astype(jnp.float32)
    idxf = jnp.min(jnp.where(d == minv, iota_f, jnp.float32(_K)),
                   axis=1, keepdims=True)
    oh = (iota_f == idxf).astype(jnp.float32)
    zq = jax.lax.dot_general(
        oh, e, (((1,), (0,)), ((), ())),
        preferred_element_type=jnp.float32)
    zq_ref[...] = zb + (zq - zb)
    idx_ref[...] = idxf.astype(jnp.int32)
    dsum_ref[...] = jnp.sum(minv).reshape(1, 1, 1)


def kernel(z, embedding):
    zp = jnp.transpose(z, (0, 2, 3, 1))
    z_flat = zp.reshape(_N_TOK, _C)
    e2 = jnp.sum(embedding ** 2, axis=1).reshape(1, _K)
    n_blocks = _N_TOK // _TB
    zq_flat, idx, dsum = pl.pallas_call(
        _vq_body,
        grid=(n_blocks,),
        in_specs=[
            pl.BlockSpec((_TB, _C), lambda i: (i, 0)),
            pl.BlockSpec((_K, _C), lambda i: (0, 0)),
            pl.BlockSpec((1, _K), lambda i: (0, 0)),
        ],
        out_specs=[
            pl.BlockSpec((_TB, _C), lambda i: (i, 0)),
            pl.BlockSpec((_TB, 1), lambda i: (i, 0)),
            pl.BlockSpec((1, 1, 1), lambda i: (i, 0, 0)),
        ],
        out_shape=[
            jax.ShapeDtypeStruct((_N_TOK, _C), jnp.float32),
            jax.ShapeDtypeStruct((_N_TOK, 1), jnp.int32),
            jax.ShapeDtypeStruct((n_blocks, 1, 1), jnp.float32),
        ],
        compiler_params=pltpu.CompilerParams(
            dimension_semantics=("parallel",)),
    )(z_flat, embedding, e2)
    min_encoding_indices = idx.reshape(_N_TOK)
    loss = 1.25 * jnp.sum(dsum) / (_N_TOK * _C)
    z_q = jnp.transpose(zq_flat.reshape(zp.shape), (0, 3, 1, 2))
    return (z_q, min_encoding_indices, loss)


# speedup vs baseline: 1.0572x; 1.0572x over previous
import jax
import jax.numpy as jnp
from jax.experimental import pallas as pl
from jax.experimental.pallas import tpu as pltpu

_K = 1024
_C = 256
_TB = 1024
_N_TOK = 8192


def _vq_body(z_ref, e_ref, e2_ref, zq_ref, idx_ref, dsum_ref):
    zb = z_ref[...]
    e = e_ref[...]
    s = jnp.sum(zb * zb, axis=1, keepdims=True)
    e2 = e2_ref[0, :]
    m = jax.lax.dot_general(
        zb, e, (((1,), (1,)), ((), ())),
        preferred_element_type=jnp.float32)
    d = (s + e2[None, :]) - 2.0 * m
    minv = jnp.min(d, axis=1, keepdims=True)
    iota_f = jax.lax.broadcasted_iota(
        jnp.int32, d.shape, 1).astype(jnp.float32)
    idxf = jnp.min(jnp.where(d == minv, iota_f, jnp.float32(_K)),
                   axis=1, keepdims=True)
    idx = idxf[:, 0].astype(jnp.int32)
    oh = (iota_f == idxf).astype(jnp.float32)
    zq = jax.lax.dot_general(
        oh, e, (((1,), (0,)), ((), ())),
        preferred_element_type=jnp.float32)
    zq_ref[...] = zb + (zq - zb)
    idx_ref[...] = idx.reshape(1, 1, _TB)
    dsum_ref[...] = jnp.sum(minv).reshape(1, 1, 1)


def kernel(z, embedding):
    zp = jnp.transpose(z, (0, 2, 3, 1))
    z_flat = zp.reshape(_N_TOK, _C)
    e2 = jnp.sum(embedding ** 2, axis=1).reshape(1, _K)
    n_blocks = _N_TOK // _TB
    zq_flat, idx, dsum = pl.pallas_call(
        _vq_body,
        grid=(n_blocks,),
        in_specs=[
            pl.BlockSpec((_TB, _C), lambda i: (i, 0)),
            pl.BlockSpec((_K, _C), lambda i: (0, 0)),
            pl.BlockSpec((1, _K), lambda i: (0, 0)),
        ],
        out_specs=[
            pl.BlockSpec((_TB, _C), lambda i: (i, 0)),
            pl.BlockSpec((1, 1, _TB), lambda i: (i, 0, 0)),
            pl.BlockSpec((1, 1, 1), lambda i: (i, 0, 0)),
        ],
        out_shape=[
            jax.ShapeDtypeStruct((_N_TOK, _C), jnp.float32),
            jax.ShapeDtypeStruct((n_blocks, 1, _TB), jnp.int32),
            jax.ShapeDtypeStruct((n_blocks, 1, 1), jnp.float32),
        ],
        compiler_params=pltpu.CompilerParams(
            dimension_semantics=("parallel",)),
    )(z_flat, embedding, e2)
    min_encoding_indices = idx.reshape(_N_TOK)
    loss = 1.25 * jnp.sum(dsum) / (_N_TOK * _C)
    z_q = jnp.transpose(zq_flat.reshape(zp.shape), (0, 3, 1, 2))
    return (z_q, min_encoding_indices, loss)
